# baseline (speedup 1.0000x reference)
import functools

import jax
import jax.numpy as jnp
from jax import lax
from jax.experimental import pallas as pl
from jax.experimental.pallas import tpu as pltpu
from jax.experimental.pallas import tpu_sc as plsc

NC = 2
NS = 16
NW = NC * NS
L = 16

B = 64
SEQ = 512
H = 768
HC = H // L
TOK = B * SEQ
TPW = TOK // NW
CH = 32
NCH = TPW // CH
CPR = SEQ // CH
TG = 8
PAD_ID = 1
MAXPOS = 514
EPS = 1e-5


def _pipeline(ids_ref, word_ref, pos_ref, tt_ref, g_ref, b_ref, out_ref,
              idx_w, idx_p, bufs_a, bufs_b, os_buf, tt_v, g_v, b_v,
              sems_a, sems_b, sem_o):
  wid = lax.axis_index("s") * NC + lax.axis_index("c")
  pltpu.sync_copy(tt_ref, tt_v)
  pltpu.sync_copy(g_ref, g_v)
  pltpu.sync_copy(b_ref, b_v)
  base = wid * TPW

  def tok0_of(c):
    return base + c * CH

  def prep(c, carry_k, s):
    pltpu.sync_copy(ids_ref.at[pl.ds(tok0_of(c), CH)], idx_w[s])
    carry_k = jnp.where(c % CPR == 0, jnp.int32(0), carry_k)

    def pos_loop(j, k):
      ids16 = idx_w[s][pl.ds(j * L, L)]
      m = jnp.where(ids16 != PAD_ID, jnp.int32(1), jnp.int32(0))
      cs = jnp.cumsum(m) + k
      idx_p[s][pl.ds(j * L, L)] = jnp.minimum(cs * m + 1,
                                              jnp.int32(MAXPOS - 1))
      return jnp.max(cs)

    return lax.fori_loop(0, CH // L, pos_loop, carry_k)

  def gather_a(s):
    return pltpu.make_async_copy(word_ref.at[idx_w[s]], bufs_a[s],
                                 sems_a[s])

  def gather_b(s):
    return pltpu.make_async_copy(pos_ref.at[idx_p[s]], bufs_b[s],
                                 sems_b[s])

  def out_copy(c):
    return pltpu.make_async_copy(os_buf,
                                 out_ref.at[pl.ds(tok0_of(c), CH)],
                                 sem_o)

  def ln_chunk(s, c):
    buf_a = bufs_a[s]
    buf_b = bufs_b[s]
    for grp in range(CH // TG):
      t0 = grp * TG

      def p1(j, carry):
        carry = list(carry)
        sl = pl.ds(j * L, L)
        ttj = tt_v[sl]
        for t in range(TG):
          d = buf_a[t0 + t, sl] + buf_b[t0 + t, sl] + ttj
          buf_a[t0 + t, sl] = d
          carry[2 * t] = carry[2 * t] + d
          carry[2 * t + 1] = carry[2 * t + 1] + d * d
        return tuple(carry)

      z = jnp.zeros((L,), jnp.float32)
      carry = lax.fori_loop(0, HC, p1, (z,) * (2 * TG))

      scales = []
      for t in range(TG):
        mean = jnp.sum(carry[2 * t]) * (1.0 / H)
        ex2 = jnp.sum(carry[2 * t + 1]) * (1.0 / H)
        x = (ex2 - mean * mean) + EPS
        i = lax.bitcast_convert_type(x, jnp.int32)
        i = jnp.int32(0x5F3759DF) - lax.shift_right_logical(i, 1)
        y = lax.bitcast_convert_type(i, jnp.float32)
        y = y * (1.5 - 0.5 * x * y * y)
        y = y * (1.5 - 0.5 * x * y * y)
        y = y * (1.5 - 0.5 * x * y * y)
        scales.append((y, mean * y))

      if grp == 0:
        out_copy(jnp.maximum(c - 1, 0)).wait()

      def p2(j, _):
        sl = pl.ds(j * L, L)
        gj = g_v[sl]
        bj = b_v[sl]
        for t in range(TG):
          d = buf_a[t0 + t, sl]
          os_buf[t0 + t, sl] = (d * scales[t][0] - scales[t][1]) * gj + bj
        return 0

      lax.fori_loop(0, HC, p2, 0)

  carry_k = prep(0, jnp.int32(0), 0)
  gather_a(0).start()
  gather_b(0).start()
  carry_k = prep(1, carry_k, 1)
  gather_a(1).start()
  gather_b(1).start()
  out_copy(0).start()

  def phase(c, carry_k, s):
    gather_a(s).wait()
    gather_b(s).wait()
    ln_chunk(s, c)
    out_copy(c).start()
    c_next = jnp.minimum(c + 2, NCH - 1)
    carry_k = prep(c_next, carry_k, s)
    gather_a(s).start()
    gather_b(s).start()
    return carry_k

  def body_i(i, carry_k):
    c = 2 * i
    carry_k = phase(c, carry_k, 0)
    carry_k = phase(c + 1, carry_k, 1)
    return carry_k

  lax.fori_loop(0, NCH // 2, body_i, carry_k)

  out_copy(NCH - 1).wait()
  gather_a(0).wait()
  gather_b(0).wait()
  gather_a(1).wait()
  gather_b(1).wait()


@functools.partial(
    pl.kernel,
    out_type=jax.ShapeDtypeStruct((TOK, H), jnp.float32),
    mesh=plsc.VectorSubcoreMesh(
        core_axis_name="c", subcore_axis_name="s",
        num_cores=NC, num_subcores=NS),
    compiler_params=pltpu.CompilerParams(needs_layout_passes=False),
    scratch_types=(
        [pltpu.VMEM((CH,), jnp.int32) for _ in range(2)]
        + [pltpu.VMEM((CH,), jnp.int32) for _ in range(2)]
        + [pltpu.VMEM((CH, H), jnp.float32) for _ in range(2)]
        + [pltpu.VMEM((CH, H), jnp.float32) for _ in range(2)]
        + [pltpu.VMEM((CH, H), jnp.float32)]
        + [pltpu.VMEM((H,), jnp.float32) for _ in range(3)]
        + [pltpu.SemaphoreType.DMA for _ in range(5)]
    ),
)
def _sc_embed_ln(ids_ref, word_ref, pos_ref, tt_ref, g_ref, b_ref, out_ref,
                 *scratch):
  idx_w = list(scratch[0:2])
  idx_p = list(scratch[2:4])
  bufs_a = list(scratch[4:6])
  bufs_b = list(scratch[6:8])
  os_buf = scratch[8]
  tt_v, g_v, b_v = scratch[9:12]
  sems_a = list(scratch[12:14])
  sems_b = list(scratch[14:16])
  sem_o = scratch[16]
  _pipeline(ids_ref, word_ref, pos_ref, tt_ref, g_ref, b_ref, out_ref,
            idx_w, idx_p, bufs_a, bufs_b, os_buf, tt_v, g_v, b_v,
            sems_a, sems_b, sem_o)


@jax.jit
def kernel(input_ids, word_embeddings, token_type_embeddings,
           position_embeddings, ln_gamma, ln_beta):
  ids = input_ids.reshape(TOK).astype(jnp.int32)
  tt_row = token_type_embeddings.reshape(H)
  out = _sc_embed_ln(ids, word_embeddings, position_embeddings,
                     tt_row, ln_gamma, ln_beta)
  return out.reshape(B, SEQ, H)


# speedup vs baseline: 1.3292x; 1.3292x over previous
import functools

import jax
import jax.numpy as jnp
from jax import lax
from jax.experimental import pallas as pl
from jax.experimental.pallas import tpu as pltpu
from jax.experimental.pallas import tpu_sc as plsc

NC = 2
NS = 16
NW = NC * NS
L = 16

B = 64
SEQ = 512
H = 768
HC = H // L
TOK = B * SEQ
TPW = TOK // NW
CH = 32
NCH = TPW // CH
CPR = SEQ // CH
TG = 8
UNROLL = 1
PAD_ID = 1
MAXPOS = 514
WLEN = CH + 8
SHIFT = 2
EPS = 1e-5


def _body(ids_ref, word_ref, pos_ref, tt_ref, g_ref, b_ref,
          out_ref,
          idx_w0, idx_p0, idx_w1, idx_p1, a0, b0, a1, b1,
          tt_v, g_v, b_v,
          sem_a0, sem_b0, sem_a1, sem_b1, sem_o0, sem_o1, sem_f):
  sid = lax.axis_index("s")
  wid = sid * NC + lax.axis_index("c")
  pltpu.sync_copy(tt_ref, tt_v)
  pltpu.sync_copy(g_ref, g_v)
  pltpu.sync_copy(b_ref, b_v)

  base = wid * TPW

  def tok0_of(c):
    return base + c * CH

  def prep(c, carry_k, idx_w, idx_p):
    pltpu.sync_copy(ids_ref.at[pl.ds(tok0_of(c), CH)], idx_w)
    carry_k = jnp.where(c % CPR == 0, jnp.int32(0), carry_k)
    k_in = carry_k

    def pos_loop(j, k):
      ids16 = idx_w[pl.ds(j * L, L)]
      m = jnp.where(ids16 != PAD_ID, jnp.int32(1), jnp.int32(0))
      cs = jnp.cumsum(m) + k
      idx_p[pl.ds(j * L, L)] = cs * m + 1
      return jnp.max(cs)

    return lax.fori_loop(0, CH // L, pos_loop, carry_k)

  def gather_a(idx_w, buf, sem):
    return pltpu.make_async_copy(word_ref.at[idx_w], buf, sem)

  def gather_b(idx_p, buf, sem):
    return pltpu.make_async_copy(pos_ref.at[idx_p], buf, sem)

  def out_copy(c, buf, sem):
    return pltpu.make_async_copy(buf, out_ref.at[pl.ds(tok0_of(c), CH)], sem)

  def ln_pass1(buf_a, buf_b):
    scales = []
    z = jnp.zeros((L,), jnp.float32)
    for grp in range(CH // TG):
      t0 = grp * TG

      def p1(j, carry):
        carry = list(carry)
        sl = pl.ds(j * L, L)
        ttj = tt_v[sl]
        for t in range(TG):
          d = buf_a[t0 + t, sl] + buf_b[t0 + t, sl] + ttj
          buf_a[t0 + t, sl] = d
          carry[2 * t] = carry[2 * t] + d
          carry[2 * t + 1] = carry[2 * t + 1] + d * d
        return tuple(carry)

      carry = lax.fori_loop(0, HC, p1, (z,) * (2 * TG))

      for t in range(TG):
        mean = jnp.sum(carry[2 * t]) * (1.0 / H)
        ex2 = jnp.sum(carry[2 * t + 1]) * (1.0 / H)
        x = (ex2 - mean * mean) + EPS
        i = lax.bitcast_convert_type(x, jnp.int32)
        i = jnp.int32(0x5F3759DF) - lax.shift_right_logical(i, 1)
        y = lax.bitcast_convert_type(i, jnp.float32)
        y = y * (1.5 - 0.5 * x * y * y)
        y = y * (1.5 - 0.5 * x * y * y)
        y = y * (1.5 - 0.5 * x * y * y)
        scales.append((y, mean * y))
    return scales

  def ln_pass2(buf_a, scales):
    for grp in range(CH // TG):
      t0 = grp * TG

      def p2(j, _):
        sl = pl.ds(j * L, L)
        gj = g_v[sl]
        bj = b_v[sl]
        for t in range(TG):
          d = buf_a[t0 + t, sl]
          y, mr = scales[t0 + t]
          buf_a[t0 + t, sl] = (d * y - mr) * gj + bj
        return 0

      lax.fori_loop(0, HC, p2, 0)

  carry_k = prep(0, jnp.int32(0), idx_w0, idx_p0)
  gather_a(idx_w0, a0, sem_a0).start()
  gather_b(idx_p0, b0, sem_b0).start()
  carry_k = prep(1, carry_k, idx_w1, idx_p1)
  gather_b(idx_p1, b1, sem_b1).start()
  out_copy(1, a1, sem_o1).start()

  idx_ws = (idx_w0, idx_w1)
  idx_ps = (idx_p0, idx_p1)
  bufs_a = (a0, a1)
  bufs_b = (b0, b1)
  sems_a = (sem_a0, sem_a1)
  sems_b = (sem_b0, sem_b1)
  sems_o = (sem_o0, sem_o1)

  def phase(c, carry_k, s):
    o = 1 - s
    gather_a(idx_ws[s], bufs_a[s], sems_a[s]).wait()
    gather_b(idx_ps[s], bufs_b[s], sems_b[s]).wait()
    scales = ln_pass1(bufs_a[s], bufs_b[s])
    out_copy(jnp.maximum(c - 1, 1 - c), bufs_a[o], sems_o[o]).wait()
    gather_a(idx_ws[o], bufs_a[o], sems_a[o]).start()
    c_next = jnp.minimum(c + 2, NCH - 1)
    carry_k = prep(c_next, carry_k, idx_ws[s], idx_ps[s])
    gather_b(idx_ps[s], bufs_b[s], sems_b[s]).start()
    ln_pass2(bufs_a[s], scales)
    out_copy(c, bufs_a[s], sems_o[s]).start()
    return carry_k

  def body_i(i, carry_k):
    c = 2 * i
    carry_k = phase(c, carry_k, 0)
    carry_k = phase(c + 1, carry_k, 1)
    return carry_k

  lax.fori_loop(0, NCH // 2, body_i, carry_k)

  out_copy(NCH - 1, a1, sem_o1).wait()
  gather_a(idx_w0, a0, sem_a0).wait()
  gather_b(idx_p0, b0, sem_b0).wait()
  gather_b(idx_p1, b1, sem_b1).wait()


@functools.partial(
    pl.kernel,
    out_type=jax.ShapeDtypeStruct((TOK, H), jnp.float32),
    mesh=plsc.VectorSubcoreMesh(
        core_axis_name="c", subcore_axis_name="s",
        num_cores=NC, num_subcores=NS),
    compiler_params=pltpu.CompilerParams(needs_layout_passes=False),
    scratch_types=[
        pltpu.VMEM((CH,), jnp.int32),
        pltpu.VMEM((CH,), jnp.int32),
        pltpu.VMEM((CH,), jnp.int32),
        pltpu.VMEM((CH,), jnp.int32),
        pltpu.VMEM((CH, H), jnp.float32),
        pltpu.VMEM((CH, H), jnp.float32),
        pltpu.VMEM((CH, H), jnp.float32),
        pltpu.VMEM((CH, H), jnp.float32),
        pltpu.VMEM((H,), jnp.float32),
        pltpu.VMEM((H,), jnp.float32),
        pltpu.VMEM((H,), jnp.float32),
        pltpu.SemaphoreType.DMA,
        pltpu.SemaphoreType.DMA,
        pltpu.SemaphoreType.DMA,
        pltpu.SemaphoreType.DMA,
        pltpu.SemaphoreType.DMA,
        pltpu.SemaphoreType.DMA,
        pltpu.SemaphoreType.DMA,
    ],
)
def _sc_embed_ln(ids_ref, word_ref, pos_ref, tt_ref, g_ref,
                 b_ref, out_ref,
                 idx_w0, idx_p0, idx_w1, idx_p1, a0, b0, a1, b1,
                 tt_v, g_v, b_v,
                 sem_a0, sem_b0, sem_a1, sem_b1, sem_o0, sem_o1, sem_f):
  _body(ids_ref, word_ref, pos_ref, tt_ref, g_ref, b_ref,
        out_ref,
        idx_w0, idx_p0, idx_w1, idx_p1, a0, b0, a1, b1,
        tt_v, g_v, b_v,
        sem_a0, sem_b0, sem_a1, sem_b1, sem_o0, sem_o1, sem_f)


@jax.jit
def kernel(input_ids, word_embeddings, token_type_embeddings,
           position_embeddings, ln_gamma, ln_beta):
  ids = input_ids.reshape(TOK).astype(jnp.int32)
  tt_row = token_type_embeddings.reshape(H)
  out = _sc_embed_ln(ids, word_embeddings, position_embeddings,
                     tt_row, ln_gamma, ln_beta)
  return out.reshape(B, SEQ, H)
